# baseline (speedup 1.0000x reference)
import functools

import jax
import jax.numpy as jnp
from jax import lax
from jax.experimental import pallas as pl
from jax.experimental.pallas import tpu as pltpu
from jax.experimental.pallas import tpu_sc as plsc

B = 4
F = 128
C = 2 * F
H = 256
W = 256
L = 16
NW = 32
CPW = C // NW
HB = 128
VPR = W // L


def _fill_col(row_v, buf):
    def body(r, _):
        for j in range(VPR):
            buf[r, pl.ds(j * L, L)] = row_v[pl.ds(j * L, L)]
        return 0

    lax.fori_loop(0, HB, body, 0)


def _fill_row(row_v, buf, h0):
    def body(g, _):
        vals = row_v[pl.ds(h0 + g * L, L)]
        for lane in range(L):
            v = jnp.full((L,), vals[lane], jnp.float32)
            for j in range(VPR):
                buf[g * L + lane, pl.ds(j * L, L)] = v
        return 0

    lax.fori_loop(0, HB // L, body, 0)


def _sc_body(colT_hbm, rowT_hbm, out_hbm, stage_v, buf0, buf1, sem0, sem1):
    cid = lax.axis_index("c")
    sid = lax.axis_index("s")
    wid = sid * 2 + cid
    ch_base = wid * CPW

    bufs = (buf0, buf1)
    sems = (sem0, sem1)
    is_col = ch_base < F

    @pl.when(is_col)
    def _():
        def body(t, _):
            pending = []
            for q in range(2):
                ch = ch_base + 2 * t + q
                pltpu.sync_copy(colT_hbm.at[ch], stage_v)
                _fill_col(stage_v, bufs[q])
                for b in range(B):
                    for half in range(2):
                        pending.append(
                            pltpu.async_copy(
                                bufs[q],
                                out_hbm.at[b, ch, pl.ds(half * HB, HB)],
                                sems[q],
                            )
                        )
            for cp in pending:
                cp.wait()
            return 0

        lax.fori_loop(0, CPW // 2, body, 0)

    @pl.when(jnp.logical_not(is_col))
    def _():
        def body(t, _):
            pending = []
            for q in range(2):
                u = 2 * t + q
                ch = ch_base + u // 2
                f = ch - F
                h0 = (u % 2) * HB
                pltpu.sync_copy(rowT_hbm.at[f], stage_v)
                _fill_row(stage_v, bufs[q], h0)
                for b in range(B):
                    pending.append(
                        pltpu.async_copy(
                            bufs[q], out_hbm.at[b, ch, pl.ds(h0, HB)], sems[q]
                        )
                    )
            for cp in pending:
                cp.wait()
            return 0

        lax.fori_loop(0, CPW, body, 0)


def kernel(bev_mask, row_embed, col_embed):
    colT = col_embed.T
    rowT = row_embed.T

    run = pl.kernel(
        _sc_body,
        mesh=plsc.VectorSubcoreMesh(core_axis_name="c", subcore_axis_name="s"),
        out_type=jax.ShapeDtypeStruct((B, C, H, W), jnp.float32),
        scratch_types=[
            pltpu.VMEM((W,), jnp.float32),
            pltpu.VMEM((HB, W), jnp.float32),
            pltpu.VMEM((HB, W), jnp.float32),
            pltpu.SemaphoreType.DMA,
            pltpu.SemaphoreType.DMA,
        ],
    )
    return run(colT, rowT)


# speedup vs baseline: 1.1527x; 1.1527x over previous
import functools

import jax
import jax.numpy as jnp
from jax import lax
from jax.experimental import pallas as pl
from jax.experimental.pallas import tpu as pltpu
from jax.experimental.pallas import tpu_sc as plsc

B = 4
F = 128
C = 2 * F
H = 256
W = 256
L = 16
NW = 32
CPW = C // NW
HB = 128
VPR = W // L


def _fill_col(row_v, buf):
    def body(r, _):
        for j in range(VPR):
            buf[r, pl.ds(j * L, L)] = row_v[pl.ds(j * L, L)]
        return 0

    lax.fori_loop(0, HB, body, 0)


def _fill_row(row_v, buf, h0):
    def body(g, _):
        vals = row_v[pl.ds(h0 + g * L, L)]
        for lane in range(L):
            v = jnp.full((L,), vals[lane], jnp.float32)
            for j in range(VPR):
                buf[g * L + lane, pl.ds(j * L, L)] = v
        return 0

    lax.fori_loop(0, HB // L, body, 0)


def _sc_body(colT_hbm, rowT_hbm, out_hbm, stage_v, buf0, buf1, sem0, sem1):
    cid = lax.axis_index("c")
    sid = lax.axis_index("s")
    wid = sid * 2 + cid
    ch_base = wid * CPW

    bufs = (buf0, buf1)
    sems = (sem0, sem1)
    is_col = ch_base < F

    def drain(q, count):
        for _ in range(count):
            pltpu.make_async_copy(colT_hbm, bufs[q], sems[q]).wait()

    @pl.when(is_col)
    def _():
        def body(t, _):
            for q in range(2):
                pl.when(t > 0)(lambda: drain(q, 2 * B))
                ch = ch_base + 2 * t + q
                pltpu.sync_copy(colT_hbm.at[ch], stage_v)
                _fill_col(stage_v, bufs[q])
                for b in range(B):
                    for half in range(2):
                        pltpu.async_copy(
                            bufs[q],
                            out_hbm.at[b, ch, pl.ds(half * HB, HB)],
                            sems[q],
                        )
            return 0

        lax.fori_loop(0, CPW // 2, body, 0)
        for q in range(2):
            drain(q, 2 * B)

    @pl.when(jnp.logical_not(is_col))
    def _():
        def body(t, _):
            for q in range(2):
                pl.when(t > 0)(lambda: drain(q, B))
                u = 2 * t + q
                ch = ch_base + u // 2
                f = ch - F
                h0 = (u % 2) * HB
                pltpu.sync_copy(rowT_hbm.at[f], stage_v)
                _fill_row(stage_v, bufs[q], h0)
                for b in range(B):
                    pltpu.async_copy(
                        bufs[q], out_hbm.at[b, ch, pl.ds(h0, HB)], sems[q]
                    )
            return 0

        lax.fori_loop(0, CPW, body, 0)
        for q in range(2):
            drain(q, B)


def kernel(bev_mask, row_embed, col_embed):
    colT = col_embed.T
    rowT = row_embed.T

    run = pl.kernel(
        _sc_body,
        mesh=plsc.VectorSubcoreMesh(core_axis_name="c", subcore_axis_name="s"),
        out_type=jax.ShapeDtypeStruct((B, C, H, W), jnp.float32),
        scratch_types=[
            pltpu.VMEM((W,), jnp.float32),
            pltpu.VMEM((HB, W), jnp.float32),
            pltpu.VMEM((HB, W), jnp.float32),
            pltpu.SemaphoreType.DMA,
            pltpu.SemaphoreType.DMA,
        ],
    )
    return run(colT, rowT)
